# baseline (speedup 1.0000x reference)
import jax
import jax.numpy as jnp
from jax import lax
from jax.experimental import pallas as pl
from jax.experimental.pallas import tpu as pltpu
from jax.experimental.pallas import tpu_sc as plsc

DIV_GUARD = 1e-05

NC, NS, L = 2, 16, 16
NW = NC * NS

B, F, T = 16, 512, 2048
ROWS = B * F
RPW = ROWS // NW
GR = 16
NG = RPW // GR
HG = GR // 2
CU = 8
TV = T // L


def _lane_shuffle(v, perm):
    dnums = lax.GatherDimensionNumbers(
        offset_dims=(), collapsed_slice_dims=(0,), start_index_map=(0,)
    )
    return lax.gather(
        v, perm[:, None], dnums, (1,),
        mode=lax.GatherScatterMode.PROMISE_IN_BOUNDS,
    )


def _sc_body(x_hbm, sl_hbm, out_hbm, sl_v, bit_v, in0, in1, o0, o1,
             si0, si1, so0, so1):
    wid = lax.axis_index("s") * NC + lax.axis_index("c")
    b = wid // (NW // B)
    pltpu.sync_copy(sl_hbm, sl_v)
    lanes = lax.iota(jnp.int32, L)
    zeros = jnp.zeros((L,), jnp.float32)
    slv = sl_v[...]
    n_i = jnp.int32(0)
    for j in range(L):
        n_i = jnp.where(b == j, slv[j], n_i)
    n_f = n_i.astype(jnp.float32)
    cb = n_i // CU
    nv = n_i // L
    base = wid * RPW
    ins = (in0, in1)
    outs = (o0, o1)
    sis = (si0, si1)
    sos = (so0, so1)

    def start_in(k, g):
        pltpu.make_async_copy(
            x_hbm.at[pl.ds((base + g * GR) * T, GR * T)], ins[k], sis[k]
        ).start()

    def wait_in(k):
        pltpu.make_async_copy(x_hbm.at[pl.ds(0, GR * T)], ins[k], sis[k]).wait()

    def start_out(h, g):
        r0 = base + g * GR + h * HG
        pltpu.make_async_copy(
            outs[h], out_hbm.at[pl.ds(r0 * T, HG * T)], sos[h]).start()

    def wait_out(h):
        pltpu.make_async_copy(
            outs[h], out_hbm.at[pl.ds(0, HG * T)], sos[h]).wait()

    def zf(jv, _):
        for r in range(HG):
            o0[pl.ds(r * T + jv * L, L)] = zeros
            o1[pl.ds(r * T + jv * L, L)] = zeros
        return 0

    lax.fori_loop(nv + 1, TV, zf, 0)

    start_in(0, 0)
    start_in(1, 1)

    def group_body(gi, _):
        for k in range(2):
            g = 2 * gi + k
            wait_in(k)
            ibuf = ins[k]

            rowbase = lanes * T

            def p1(jb, carry):
                s, ss, idx = carry
                for _u in range(CU):
                    v = plsc.load_gather(ibuf, [idx])
                    idx = idx + 1
                    s = s + v
                    ss = ss + v * v
                return s, ss, idx

            s, ss, idx = lax.fori_loop(0, cb, p1, (zeros, zeros, rowbase))
            for u in range(CU):
                c = cb * CU + u
                v = plsc.load_gather(ibuf, [idx])
                idx = idx + 1
                v = jnp.where(c < n_i, v, 0.0)
                s = s + v
                ss = ss + v * v

            mean = s / n_f
            var = (ss - n_f * mean * mean) / (n_f - 1.0)
            var = jnp.maximum(var, 1e-30)
            bit_v.bitcast(jnp.float32)[0, :] = var
            iv = bit_v[0, :]
            iv = 0x5F3759DF - lax.shift_right_logical(iv, 1)
            bit_v[0, :] = iv
            y = bit_v.bitcast(jnp.float32)[0, :]
            for _ in range(3):
                y = y * (1.5 - 0.5 * var * y * y)
            inv = 1.0 / (var * y + DIV_GUARD)

            for h in range(2):
                @pl.when(g > 0)
                def _(h=h):
                    wait_out(h)

                obuf = outs[h]
                for r in range(HG):
                    row = h * HG + r
                    perm_r = jnp.full((L,), row, jnp.int32)
                    mean_r = _lane_shuffle(mean, perm_r)
                    inv_r = _lane_shuffle(inv, perm_r)
                    src0 = row * T
                    dst0 = r * T

                    def p2(jv, _, mean_r=mean_r, inv_r=inv_r,
                           src0=src0, dst0=dst0, obuf=obuf):
                        v = ibuf[pl.ds(src0 + jv * L, L)]
                        obuf[pl.ds(dst0 + jv * L, L)] = (v - mean_r) * inv_r
                        return 0

                    lax.fori_loop(0, nv, p2, 0)
                    t = lanes + nv * L
                    v = ibuf[pl.ds(src0 + nv * L, L)]
                    w = jnp.where(t < n_i, (v - mean_r) * inv_r, 0.0)
                    obuf[pl.ds(dst0 + nv * L, L)] = w

                start_out(h, g)

            @pl.when(g + 2 < NG)
            def _(k=k, g=g):
                start_in(k, g + 2)
        return 0

    lax.fori_loop(0, NG // 2, group_body, 0)
    wait_out(0)
    wait_out(1)


def kernel(x, seq_lens):
    Bx, Fx, Tx = x.shape
    x2 = x.reshape(Bx * Fx * Tx)
    sl = seq_lens.astype(jnp.int32)
    mesh = plsc.VectorSubcoreMesh(
        core_axis_name="c", subcore_axis_name="s", num_cores=NC, num_subcores=NS
    )
    out = pl.kernel(
        _sc_body,
        out_type=jax.ShapeDtypeStruct((ROWS * T,), jnp.float32),
        mesh=mesh,
        compiler_params=pltpu.CompilerParams(
            use_tc_tiling_on_sc=False, needs_layout_passes=False
        ),
        scratch_types=[
            pltpu.VMEM((L,), jnp.int32),
            pltpu.VMEM((1, L), jnp.int32),
            pltpu.VMEM((GR * T,), jnp.float32),
            pltpu.VMEM((GR * T,), jnp.float32),
            pltpu.VMEM((HG * T,), jnp.float32),
            pltpu.VMEM((HG * T,), jnp.float32),
            pltpu.SemaphoreType.DMA,
            pltpu.SemaphoreType.DMA,
            pltpu.SemaphoreType.DMA,
            pltpu.SemaphoreType.DMA,
        ],
    )(x2, sl)
    return out.reshape(Bx, Fx, Tx)


# speedup vs baseline: 2.0137x; 2.0137x over previous
import jax
import jax.numpy as jnp
from jax import lax
from jax.experimental import pallas as pl
from jax.experimental.pallas import tpu as pltpu
from jax.experimental.pallas import tpu_sc as plsc

DIV_GUARD = 1e-05

NC, NS, L = 2, 16, 16
NW = NC * NS

B, F, T = 16, 512, 2048
ROWS = B * F
RPW = ROWS // NW
RC = 8
NCHUNK = RPW // RC
TV = T // L
UB = 8
NB = TV // UB


def _lane_shuffle(v, perm):
    dnums = lax.GatherDimensionNumbers(
        offset_dims=(), collapsed_slice_dims=(0,), start_index_map=(0,)
    )
    return lax.gather(
        v, perm[:, None], dnums, (1,),
        mode=lax.GatherScatterMode.PROMISE_IN_BOUNDS,
    )


def _row_normalize(ibuf, obuf, bit_v, r, n_i, n_f, fb, lanes, zeros):
    def p1(jb, carry):
        s, ss = carry
        for u in range(UB):
            v = ibuf[r, pl.ds((jb * UB + u) * L, L)]
            s = s + v
            ss = ss + v * v
        return s, ss

    s, ss = lax.fori_loop(0, fb, p1, (zeros, zeros))
    for u in range(UB):
        j = fb * UB + u
        t = lanes + j * L
        v = ibuf[r, pl.ds(j * L, L)]
        vm = jnp.where(t < n_i, v, 0.0)
        s = s + vm
        ss = ss + vm * vm
    for sh in (8, 4, 2, 1):
        perm = lanes ^ sh
        s = s + _lane_shuffle(s, perm)
        ss = ss + _lane_shuffle(ss, perm)
    mean_v = s / n_f
    var_v = (ss - n_f * mean_v * mean_v) / (n_f - 1.0)
    var_v = jnp.maximum(var_v, 1e-30)
    bit_v.bitcast(jnp.float32)[0, :] = var_v
    iv = bit_v[0, :]
    iv = 0x5F3759DF - lax.shift_right_logical(iv, 1)
    bit_v[0, :] = iv
    y = bit_v.bitcast(jnp.float32)[0, :]
    for _ in range(3):
        y = y * (1.5 - 0.5 * var_v * y * y)
    std = var_v * y + DIV_GUARD
    inv = 1.0 / std

    def p2(jb, _):
        for u in range(UB):
            j2 = jb * UB + u
            v = ibuf[r, pl.ds(j2 * L, L)]
            obuf[r, pl.ds(j2 * L, L)] = (v - mean_v) * inv
        return 0

    lax.fori_loop(0, fb, p2, 0)
    for u in range(UB):
        j = fb * UB + u
        t = lanes + j * L
        v = ibuf[r, pl.ds(j * L, L)]
        obuf[r, pl.ds(j * L, L)] = jnp.where(t < n_i, (v - mean_v) * inv, 0.0)


def _sc_body(x_hbm, sl_hbm, out_hbm, sl_v, bit_v, in0, in1, o0, o1,
             si0, si1, so0, so1):
    wid = lax.axis_index("s") * NC + lax.axis_index("c")
    b = wid // (NW // B)
    pltpu.sync_copy(sl_hbm, sl_v)
    lanes = lax.iota(jnp.int32, L)
    zeros = jnp.zeros((L,), jnp.float32)
    slv = sl_v[...]
    n_i = jnp.int32(0)
    for j in range(L):
        n_i = jnp.where(b == j, slv[j], n_i)
    n_f = n_i.astype(jnp.float32)
    fb = n_i // (UB * L)
    base = wid * RPW
    ins = (in0, in1)
    outs = (o0, o1)
    sis = (si0, si1)
    sos = (so0, so1)

    def start_in(k, c):
        pltpu.make_async_copy(
            x_hbm.at[pl.ds(base + c * RC, RC)], ins[k], sis[k]).start()

    def wait_in(k):
        pltpu.make_async_copy(x_hbm.at[pl.ds(0, RC)], ins[k], sis[k]).wait()

    def start_out(k, c):
        pltpu.make_async_copy(
            outs[k], out_hbm.at[pl.ds(base + c * RC, RC)], sos[k]).start()

    def wait_out(k):
        pltpu.make_async_copy(
            outs[k], out_hbm.at[pl.ds(0, RC)], sos[k]).wait()

    def zf_row(r, _):
        def zf(jv, _2):
            o0[r, pl.ds(jv * L, L)] = zeros
            o1[r, pl.ds(jv * L, L)] = zeros
            return 0

        lax.fori_loop((fb + 1) * UB, TV, zf, 0)
        return 0

    lax.fori_loop(0, RC, zf_row, 0)

    start_in(0, 0)
    start_in(1, 1)

    def pair_body(g, _):
        for k in range(2):
            c = 2 * g + k
            wait_in(k)

            @pl.when(g > 0)
            def _(k=k):
                wait_out(k)

            def rows(r, _2, k=k):
                _row_normalize(ins[k], outs[k], bit_v, r, n_i, n_f, fb,
                               lanes, zeros)
                return 0

            lax.fori_loop(0, RC, rows, 0)
            start_out(k, c)

            @pl.when(c + 2 < NCHUNK)
            def _(k=k, c=c):
                start_in(k, c + 2)
        return 0

    lax.fori_loop(0, NCHUNK // 2, pair_body, 0)
    wait_out(0)
    wait_out(1)


def kernel(x, seq_lens):
    Bx, Fx, Tx = x.shape
    x2 = x.reshape(Bx * Fx, Tx)
    sl = seq_lens.astype(jnp.int32)
    mesh = plsc.VectorSubcoreMesh(
        core_axis_name="c", subcore_axis_name="s", num_cores=NC, num_subcores=NS
    )
    out = pl.kernel(
        _sc_body,
        out_type=jax.ShapeDtypeStruct((ROWS, T), jnp.float32),
        mesh=mesh,
        scratch_types=[
            pltpu.VMEM((L,), jnp.int32),
            pltpu.VMEM((1, L), jnp.int32),
            pltpu.VMEM((RC, T), jnp.float32),
            pltpu.VMEM((RC, T), jnp.float32),
            pltpu.VMEM((RC, T), jnp.float32),
            pltpu.VMEM((RC, T), jnp.float32),
            pltpu.SemaphoreType.DMA,
            pltpu.SemaphoreType.DMA,
            pltpu.SemaphoreType.DMA,
            pltpu.SemaphoreType.DMA,
        ],
    )(x2, sl)
    return out.reshape(Bx, Fx, Tx)


# speedup vs baseline: 3.6264x; 1.8008x over previous
import jax
import jax.numpy as jnp
from jax import lax
from jax.experimental import pallas as pl
from jax.experimental.pallas import tpu as pltpu
from jax.experimental.pallas import tpu_sc as plsc

DIV_GUARD = 1e-05

NC, NS, L = 2, 16, 16
NW = NC * NS

B, F, T = 16, 512, 2048
KSC = 4
SC_ROWS = KSC * F
RPW = SC_ROWS // NW
RC = 8
NCHUNK = RPW // RC
TV = T // L
UB = 8
NB = TV // UB


def _lane_shuffle(v, perm):
    dnums = lax.GatherDimensionNumbers(
        offset_dims=(), collapsed_slice_dims=(0,), start_index_map=(0,)
    )
    return lax.gather(
        v, perm[:, None], dnums, (1,),
        mode=lax.GatherScatterMode.PROMISE_IN_BOUNDS,
    )


def _row_normalize(buf, bit_v, r, n_i, n_f, fb, lanes, zeros):
    def p1(jb, carry):
        s, ss = carry
        for u in range(UB):
            v = buf[r, pl.ds((jb * UB + u) * L, L)]
            s = s + v
            ss = ss + v * v
        return s, ss

    s, ss = lax.fori_loop(0, fb, p1, (zeros, zeros))
    for u in range(UB):
        j = fb * UB + u
        t = lanes + j * L
        v = buf[r, pl.ds(j * L, L)]
        vm = jnp.where(t < n_i, v, 0.0)
        s = s + vm
        ss = ss + vm * vm
    for sh in (8, 4, 2, 1):
        perm = lanes ^ sh
        s = s + _lane_shuffle(s, perm)
        ss = ss + _lane_shuffle(ss, perm)
    mean_v = s / n_f
    var_v = (ss - n_f * mean_v * mean_v) / (n_f - 1.0)
    var_v = jnp.maximum(var_v, 1e-30)
    bit_v.bitcast(jnp.float32)[0, :] = var_v
    iv = bit_v[0, :]
    iv = 0x5F3759DF - lax.shift_right_logical(iv, 1)
    bit_v[0, :] = iv
    y = bit_v.bitcast(jnp.float32)[0, :]
    for _ in range(3):
        y = y * (1.5 - 0.5 * var_v * y * y)
    std = var_v * y + DIV_GUARD
    inv = 1.0 / std

    def p2(jb, _):
        for u in range(UB):
            j2 = jb * UB + u
            v = buf[r, pl.ds(j2 * L, L)]
            buf[r, pl.ds(j2 * L, L)] = (v - mean_v) * inv
        return 0

    lax.fori_loop(0, fb, p2, 0)
    for u in range(UB):
        j = fb * UB + u
        t = lanes + j * L
        v = buf[r, pl.ds(j * L, L)]
        buf[r, pl.ds(j * L, L)] = jnp.where(t < n_i, (v - mean_v) * inv, 0.0)

    def p3(jb, _):
        for u in range(UB):
            buf[r, pl.ds((jb * UB + u) * L, L)] = zeros
        return 0

    lax.fori_loop(fb + 1, NB, p3, 0)


def _sc_body(x_hbm, sl_hbm, out_hbm, sl_v, bit_v, buf):
    wid = lax.axis_index("s") * NC + lax.axis_index("c")
    b = wid // (NW // KSC)
    pltpu.sync_copy(sl_hbm, sl_v)
    lanes = lax.iota(jnp.int32, L)
    zeros = jnp.zeros((L,), jnp.float32)
    slv = sl_v[...]
    n_i = jnp.int32(0)
    for j in range(L):
        n_i = jnp.where(b == j, slv[j], n_i)
    n_f = n_i.astype(jnp.float32)
    fb = n_i // (UB * L)
    base = wid * RPW

    def chunk_body(c, _):
        row0 = base + c * RC
        pltpu.sync_copy(x_hbm.at[pl.ds(row0, RC)], buf)

        def rows(r, _2):
            _row_normalize(buf, bit_v, r, n_i, n_f, fb, lanes, zeros)
            return 0

        lax.fori_loop(0, RC, rows, 0)
        pltpu.sync_copy(buf, out_hbm.at[pl.ds(row0, RC)])
        return 0

    lax.fori_loop(0, NCHUNK, chunk_body, 0)


def _sc_part(x_sc, sl):
    mesh = plsc.VectorSubcoreMesh(
        core_axis_name="c", subcore_axis_name="s", num_cores=NC, num_subcores=NS
    )
    return pl.kernel(
        _sc_body,
        out_type=jax.ShapeDtypeStruct((SC_ROWS, T), jnp.float32),
        mesh=mesh,
        scratch_types=[
            pltpu.VMEM((L,), jnp.int32),
            pltpu.VMEM((1, L), jnp.int32),
            pltpu.VMEM((RC, T), jnp.float32),
        ],
    )(x_sc, sl)


def _tc_body(sl_ref, x_ref, o_ref):
    bt = pl.program_id(0)
    n = sl_ref[bt + KSC].astype(jnp.float32)
    xv = x_ref[...]
    t = jax.lax.broadcasted_iota(jnp.int32, (1, 1, T), 2)
    mask = (t < sl_ref[bt + KSC]).astype(jnp.float32)
    xm = xv * mask
    s = jnp.sum(xm, axis=2, keepdims=True)
    ss = jnp.sum(xm * xm, axis=2, keepdims=True)
    mean = s / n
    var = (ss - n * mean * mean) / (n - 1.0)
    var = jnp.maximum(var, 0.0)
    std = jnp.sqrt(var) + DIV_GUARD
    o_ref[...] = (xm - mean * mask) / std


def _tc_part(x_tc, sl):
    nb = B - KSC
    return pl.pallas_call(
        _tc_body,
        grid=(nb,),
        in_specs=[
            pl.BlockSpec(memory_space=pltpu.SMEM),
            pl.BlockSpec((1, F, T), lambda bb: (bb, 0, 0)),
        ],
        out_specs=pl.BlockSpec((1, F, T), lambda bb: (bb, 0, 0)),
        out_shape=jax.ShapeDtypeStruct((nb, F, T), x_tc.dtype),
    )(sl, x_tc)


def kernel(x, seq_lens):
    sl = seq_lens.astype(jnp.int32)
    x_sc = x[:KSC].reshape(SC_ROWS, T)
    x_tc = x[KSC:]
    sc_out = _sc_part(x_sc, sl).reshape(KSC, F, T)
    tc_out = _tc_part(x_tc, sl)
    return jnp.concatenate([sc_out, tc_out], axis=0)


# speedup vs baseline: 6.7997x; 1.8750x over previous
import jax
import jax.numpy as jnp
from jax import lax
from jax.experimental import pallas as pl
from jax.experimental.pallas import tpu as pltpu
from jax.experimental.pallas import tpu_sc as plsc

DIV_GUARD = 1e-05

NC, NS, L = 2, 16, 16
NW = NC * NS

B, F, T = 16, 512, 2048
ROWS = B * F
KSC = 2
SC_ROWS = KSC * F
RPW = SC_ROWS // NW
RC = 8
NCHUNK = RPW // RC
TV = T // L
UB = 8
NB = TV // UB


def _lane_shuffle(v, perm):
    dnums = lax.GatherDimensionNumbers(
        offset_dims=(), collapsed_slice_dims=(0,), start_index_map=(0,)
    )
    return lax.gather(
        v, perm[:, None], dnums, (1,),
        mode=lax.GatherScatterMode.PROMISE_IN_BOUNDS,
    )


def _row_normalize(buf, bit_v, r, n_i, n_f, fb, lanes, zeros):
    def p1(jb, carry):
        s, ss = carry
        for u in range(UB):
            v = buf[r, pl.ds((jb * UB + u) * L, L)]
            s = s + v
            ss = ss + v * v
        return s, ss

    s, ss = lax.fori_loop(0, fb, p1, (zeros, zeros))
    for u in range(UB):
        j = fb * UB + u
        t = lanes + j * L
        v = buf[r, pl.ds(j * L, L)]
        vm = jnp.where(t < n_i, v, 0.0)
        s = s + vm
        ss = ss + vm * vm
    for sh in (8, 4, 2, 1):
        perm = lanes ^ sh
        s = s + _lane_shuffle(s, perm)
        ss = ss + _lane_shuffle(ss, perm)
    mean_v = s / n_f
    var_v = (ss - n_f * mean_v * mean_v) / (n_f - 1.0)
    var_v = jnp.maximum(var_v, 1e-30)
    bit_v.bitcast(jnp.float32)[0, :] = var_v
    iv = bit_v[0, :]
    iv = 0x5F3759DF - lax.shift_right_logical(iv, 1)
    bit_v[0, :] = iv
    y = bit_v.bitcast(jnp.float32)[0, :]
    for _ in range(3):
        y = y * (1.5 - 0.5 * var_v * y * y)
    std = var_v * y + DIV_GUARD
    inv = 1.0 / std

    def p2(jb, _):
        for u in range(UB):
            j2 = jb * UB + u
            v = buf[r, pl.ds(j2 * L, L)]
            buf[r, pl.ds(j2 * L, L)] = (v - mean_v) * inv
        return 0

    lax.fori_loop(0, fb, p2, 0)
    for u in range(UB):
        j = fb * UB + u
        t = lanes + j * L
        v = buf[r, pl.ds(j * L, L)]
        buf[r, pl.ds(j * L, L)] = jnp.where(t < n_i, (v - mean_v) * inv, 0.0)

    def p3(jb, _):
        for u in range(UB):
            buf[r, pl.ds((jb * UB + u) * L, L)] = zeros
        return 0

    lax.fori_loop(fb + 1, NB, p3, 0)


def _sc_body(x_hbm, sl_hbm, out_hbm, sl_v, bit_v, buf):
    wid = lax.axis_index("s") * NC + lax.axis_index("c")
    b = wid // (NW // KSC)
    pltpu.sync_copy(sl_hbm, sl_v)
    lanes = lax.iota(jnp.int32, L)
    zeros = jnp.zeros((L,), jnp.float32)
    slv = sl_v[...]
    n_i = jnp.int32(0)
    for j in range(L):
        n_i = jnp.where(b == j, slv[j], n_i)
    n_f = n_i.astype(jnp.float32)
    fb = n_i // (UB * L)
    base = wid * RPW

    def chunk_body(c, _):
        row0 = base + c * RC
        pltpu.sync_copy(x_hbm.at[pl.ds(row0, RC)], buf)

        def rows(r, _2):
            _row_normalize(buf, bit_v, r, n_i, n_f, fb, lanes, zeros)
            return 0

        lax.fori_loop(0, RC, rows, 0)
        pltpu.sync_copy(buf, out_hbm.at[pl.ds(row0, RC)])
        return 0

    lax.fori_loop(0, NCHUNK, chunk_body, 0)


def _sc_part(x2, sl):
    mesh = plsc.VectorSubcoreMesh(
        core_axis_name="c", subcore_axis_name="s", num_cores=NC, num_subcores=NS
    )
    return pl.kernel(
        _sc_body,
        out_type=jax.ShapeDtypeStruct((ROWS, T), jnp.float32),
        mesh=mesh,
        scratch_types=[
            pltpu.VMEM((L,), jnp.int32),
            pltpu.VMEM((1, L), jnp.int32),
            pltpu.VMEM((RC, T), jnp.float32),
        ],
    )(x2, sl)


def _tc_body(sl_ref, x_ref, sc_ref, o_ref):
    del sc_ref
    bt = pl.program_id(0)
    n = sl_ref[bt + KSC].astype(jnp.float32)
    xv = x_ref[...]
    t = jax.lax.broadcasted_iota(jnp.int32, (1, 1, T), 2)
    mask = (t < sl_ref[bt + KSC]).astype(jnp.float32)
    xm = xv * mask
    s = jnp.sum(xm, axis=2, keepdims=True)
    ss = jnp.sum(xm * xm, axis=2, keepdims=True)
    mean = s / n
    var = (ss - n * mean * mean) / (n - 1.0)
    var = jnp.maximum(var, 0.0)
    std = jnp.sqrt(var) + DIV_GUARD
    o_ref[...] = (xm - mean * mask) / std


def _tc_part(sl, x, sc_full):
    nb = B - KSC
    return pl.pallas_call(
        _tc_body,
        grid=(nb,),
        in_specs=[
            pl.BlockSpec(memory_space=pltpu.SMEM),
            pl.BlockSpec((1, F, T), lambda bb: (bb + KSC, 0, 0)),
            pl.BlockSpec(memory_space=pl.ANY),
        ],
        out_specs=pl.BlockSpec((1, F, T), lambda bb: (bb + KSC, 0, 0)),
        out_shape=jax.ShapeDtypeStruct((B, F, T), x.dtype),
        input_output_aliases={2: 0},
    )(sl, x, sc_full)


def kernel(x, seq_lens):
    sl = seq_lens.astype(jnp.int32)
    x2 = x.reshape(ROWS, T)
    sc_full = _sc_part(x2, sl).reshape(B, F, T)
    return _tc_part(sl, x, sc_full)


# speedup vs baseline: 7.6946x; 1.1316x over previous
import jax
import jax.numpy as jnp
from jax import lax
from jax.experimental import pallas as pl
from jax.experimental.pallas import tpu as pltpu
from jax.experimental.pallas import tpu_sc as plsc

DIV_GUARD = 1e-05

NC, NS, L = 2, 16, 16
NW = NC * NS

B, F, T = 16, 512, 2048
ROWS = B * F
KSC = 1
SC_ROWS = KSC * F
RPW = SC_ROWS // NW
RC = 8
NCHUNK = RPW // RC
TV = T // L
UB = 8
NB = TV // UB


def _lane_shuffle(v, perm):
    dnums = lax.GatherDimensionNumbers(
        offset_dims=(), collapsed_slice_dims=(0,), start_index_map=(0,)
    )
    return lax.gather(
        v, perm[:, None], dnums, (1,),
        mode=lax.GatherScatterMode.PROMISE_IN_BOUNDS,
    )


def _row_normalize(buf, bit_v, r, n_i, n_f, fb, lanes, zeros):
    def p1(jb, carry):
        s, ss = carry
        for u in range(UB):
            v = buf[r, pl.ds((jb * UB + u) * L, L)]
            s = s + v
            ss = ss + v * v
        return s, ss

    s, ss = lax.fori_loop(0, fb, p1, (zeros, zeros))
    for u in range(UB):
        j = fb * UB + u
        t = lanes + j * L
        v = buf[r, pl.ds(j * L, L)]
        vm = jnp.where(t < n_i, v, 0.0)
        s = s + vm
        ss = ss + vm * vm
    for sh in (8, 4, 2, 1):
        perm = lanes ^ sh
        s = s + _lane_shuffle(s, perm)
        ss = ss + _lane_shuffle(ss, perm)
    mean_v = s / n_f
    var_v = (ss - n_f * mean_v * mean_v) / (n_f - 1.0)
    var_v = jnp.maximum(var_v, 1e-30)
    bit_v.bitcast(jnp.float32)[0, :] = var_v
    iv = bit_v[0, :]
    iv = 0x5F3759DF - lax.shift_right_logical(iv, 1)
    bit_v[0, :] = iv
    y = bit_v.bitcast(jnp.float32)[0, :]
    for _ in range(3):
        y = y * (1.5 - 0.5 * var_v * y * y)
    std = var_v * y + DIV_GUARD
    inv = 1.0 / std

    def p2(jb, _):
        for u in range(UB):
            j2 = jb * UB + u
            v = buf[r, pl.ds(j2 * L, L)]
            buf[r, pl.ds(j2 * L, L)] = (v - mean_v) * inv
        return 0

    lax.fori_loop(0, fb, p2, 0)
    for u in range(UB):
        j = fb * UB + u
        t = lanes + j * L
        v = buf[r, pl.ds(j * L, L)]
        buf[r, pl.ds(j * L, L)] = jnp.where(t < n_i, (v - mean_v) * inv, 0.0)

    def p3(jb, _):
        for u in range(UB):
            buf[r, pl.ds((jb * UB + u) * L, L)] = zeros
        return 0

    lax.fori_loop(fb + 1, NB, p3, 0)


def _sc_body(x_hbm, sl_hbm, out_hbm, sl_v, bit_v, buf):
    wid = lax.axis_index("s") * NC + lax.axis_index("c")
    b = wid // (NW // KSC)
    pltpu.sync_copy(sl_hbm, sl_v)
    lanes = lax.iota(jnp.int32, L)
    zeros = jnp.zeros((L,), jnp.float32)
    slv = sl_v[...]
    n_i = jnp.int32(0)
    for j in range(L):
        n_i = jnp.where(b == j, slv[j], n_i)
    n_f = n_i.astype(jnp.float32)
    fb = n_i // (UB * L)
    base = wid * RPW

    def chunk_body(c, _):
        row0 = base + c * RC
        pltpu.sync_copy(x_hbm.at[pl.ds(row0, RC)], buf)

        def rows(r, _2):
            _row_normalize(buf, bit_v, r, n_i, n_f, fb, lanes, zeros)
            return 0

        lax.fori_loop(0, RC, rows, 0)
        pltpu.sync_copy(buf, out_hbm.at[pl.ds(row0, RC)])
        return 0

    lax.fori_loop(0, NCHUNK, chunk_body, 0)


def _sc_part(x2, sl):
    mesh = plsc.VectorSubcoreMesh(
        core_axis_name="c", subcore_axis_name="s", num_cores=NC, num_subcores=NS
    )
    return pl.kernel(
        _sc_body,
        out_type=jax.ShapeDtypeStruct((ROWS, T), jnp.float32),
        mesh=mesh,
        scratch_types=[
            pltpu.VMEM((L,), jnp.int32),
            pltpu.VMEM((1, L), jnp.int32),
            pltpu.VMEM((RC, T), jnp.float32),
        ],
    )(x2, sl)


def _tc_body(sl_ref, x_ref, sc_ref, o_ref):
    del sc_ref
    bt = pl.program_id(0)
    n = sl_ref[bt + KSC].astype(jnp.float32)
    xv = x_ref[...]
    t = jax.lax.broadcasted_iota(jnp.int32, (1, 1, T), 2)
    mask = (t < sl_ref[bt + KSC]).astype(jnp.float32)
    xm = xv * mask
    s = jnp.sum(xm, axis=2, keepdims=True)
    ss = jnp.sum(xm * xm, axis=2, keepdims=True)
    mean = s / n
    var = (ss - n * mean * mean) / (n - 1.0)
    var = jnp.maximum(var, 0.0)
    std = jnp.sqrt(var) + DIV_GUARD
    o_ref[...] = (xm - mean * mask) / std


def _tc_part(sl, x, sc_full):
    nb = B - KSC
    return pl.pallas_call(
        _tc_body,
        grid=(nb,),
        in_specs=[
            pl.BlockSpec(memory_space=pltpu.SMEM),
            pl.BlockSpec((1, F, T), lambda bb: (bb + KSC, 0, 0)),
            pl.BlockSpec(memory_space=pl.ANY),
        ],
        out_specs=pl.BlockSpec((1, F, T), lambda bb: (bb + KSC, 0, 0)),
        out_shape=jax.ShapeDtypeStruct((B, F, T), x.dtype),
        input_output_aliases={2: 0},
    )(sl, x, sc_full)


def kernel(x, seq_lens):
    sl = seq_lens.astype(jnp.int32)
    x2 = x.reshape(ROWS, T)
    sc_full = _sc_part(x2, sl).reshape(B, F, T)
    return _tc_part(sl, x, sc_full)
